# baseline (speedup 1.0000x reference)
import functools

import jax
import jax.numpy as jnp
from jax import lax
from jax.experimental import pallas as pl
from jax.experimental.pallas import tpu as pltpu
from jax.experimental.pallas import tpu_sc as plsc

N = 10000
E = 320000
D = 128

NC, NS, L = 2, 16, 16
NW = NC * NS
B = 128
CH = 80
EP = NW * CH * B
NP = 10240
RPT = NP // NS
HD2 = D // 2


_sc_mesh = plsc.VectorSubcoreMesh(core_axis_name="c", subcore_axis_name="s")
_sc_params = pltpu.CompilerParams(needs_layout_passes=False,
                                  use_tc_tiling_on_sc=False)


@functools.partial(
    pl.kernel,
    out_type=jax.ShapeDtypeStruct((NW, NP), jnp.float32),
    mesh=_sc_mesh,
    compiler_params=_sc_params,
    scratch_types=[
        pltpu.VMEM((CH, B), jnp.int32),
        pltpu.VMEM((NP,), jnp.float32),
    ],
)
def _deg_kernel(colp_hbm, degp_hbm, col_v, deg_v):
    cid = lax.axis_index("c")
    sid = lax.axis_index("s")
    wid = cid * NS + sid

    pltpu.sync_copy(colp_hbm.at[pl.ds(wid * CH, CH)], col_v)

    zeros16 = jnp.zeros((L,), jnp.float32)

    @pl.loop(0, NP // L)
    def _(i):
        deg_v[pl.ds(i * L, L)] = zeros16

    ones16 = jnp.ones((L,), jnp.float32)

    @pl.loop(0, CH)
    def _(i):
        for k in range(B // L):
            idx = col_v[i, pl.ds(k * L, L)]
            plsc.addupdate_scatter(deg_v, [idx], ones16)

    pltpu.sync_copy(deg_v, degp_hbm.at[wid])


CHT = EP // (NS * B)
CHP = 20
NB = 4
HBM_MOD = 8
HBM_CNT = 3


@functools.partial(
    pl.kernel,
    out_type=jax.ShapeDtypeStruct((NC, NP, HD2), jnp.float32),
    mesh=_sc_mesh,
    compiler_params=_sc_params,
    scratch_types=[
        pltpu.VMEM((CHP, B), jnp.int32),
        pltpu.VMEM((CHP, B), jnp.int32),
        pltpu.VMEM((CHP, B), jnp.int32),
        [pltpu.VMEM((B, HD2), jnp.float32)] * NB,
        pltpu.VMEM((16, HD2), jnp.float32),
        pltpu.VMEM_SHARED((NP, HD2), jnp.float32),
        pltpu.VMEM_SHARED((NP, HD2), jnp.float32),
        [pltpu.SemaphoreType.DMA] * NB,
        [pltpu.SemaphoreType.DMA] * NB,
    ],
)
def _msg_kernel(ysf_hbm, rowp_hbm, colp_hbm, hp_hbm,
                row_v, rowa_v, col_v, bufs, zbuf, y_s, acc_s, gsem, ssem):
    cid = lax.axis_index("c")
    sid = lax.axis_index("s")
    yoff = cid * NP

    pltpu.sync_copy(ysf_hbm.at[pl.ds(yoff + sid * RPT, RPT)],
                    y_s.at[pl.ds(sid * RPT, RPT)])

    zeros16 = jnp.zeros((L,), jnp.float32)

    @pl.loop(0, 16)
    def _(i):
        for k in range(HD2 // L):
            zbuf[i, pl.ds(k * L, L)] = zeros16

    @pl.loop(0, RPT // 16)
    def _(j):
        pltpu.sync_copy(zbuf, acc_s.at[pl.ds(sid * RPT + j * 16, 16)])

    plsc.subcore_barrier()

    def from_hbm(c):
        return lax.rem(jnp.int32(c), jnp.int32(HBM_MOD)) < HBM_CNT

    def start_gather(c, b):
        @pl.when(from_hbm(c))
        def _():
            pltpu.async_copy(ysf_hbm.at[rowa_v.at[c]], bufs[b], gsem[b])

        @pl.when(jnp.logical_not(from_hbm(c)))
        def _():
            pltpu.async_copy(y_s.at[row_v.at[c]], bufs[b], gsem[b])

    def wait_gather(c, b):
        @pl.when(from_hbm(c))
        def _():
            pltpu.make_async_copy(ysf_hbm.at[rowa_v.at[c]],
                                  bufs[b], gsem[b]).wait()

        @pl.when(jnp.logical_not(from_hbm(c)))
        def _():
            pltpu.make_async_copy(y_s.at[row_v.at[c]],
                                  bufs[b], gsem[b]).wait()

    def one_pass(base):
        pltpu.sync_copy(rowp_hbm.at[pl.ds(base, CHP)], row_v)
        pltpu.sync_copy(colp_hbm.at[pl.ds(base, CHP)], col_v)

        @pl.loop(0, CHP)
        def _(i):
            for k in range(B // L):
                rowa_v[i, pl.ds(k * L, L)] = row_v[i, pl.ds(k * L, L)] + yoff

        for b in range(NB):
            start_gather(b, b)

        @pl.loop(0, CHP, step=NB)
        def _(j):
            for b in range(NB):
                wait_gather(j + b, b)
                pltpu.async_copy(bufs[b], acc_s.at[col_v.at[j + b]],
                                 ssem[b], add=True)
            for b in range(NB):
                pltpu.make_async_copy(bufs[b], acc_s.at[col_v.at[j + b]],
                                      ssem[b]).wait()

                @pl.when(j + NB + b < CHP)
                def _():
                    start_gather(j + NB + b, b)

    for p in range(CHT // CHP):
        one_pass(sid * CHT + p * CHP)

    plsc.subcore_barrier()
    pltpu.sync_copy(acc_s.at[pl.ds(sid * RPT, RPT)],
                    hp_hbm.at[cid, pl.ds(sid * RPT, RPT)])


def _lstm_w_new(W_gcn, W_ih, W_hh, b_ih, b_hh, h0, c0):
    dn = (((1,), (1,)), ((), ()))
    gates = (lax.dot_general(W_gcn, W_ih, dn, preferred_element_type=jnp.float32)
             + lax.dot_general(h0, W_hh, dn, preferred_element_type=jnp.float32)
             + b_ih[0, :] + b_hh[0, :])
    ig = jax.nn.sigmoid(gates[:, 0:D])
    fg = jax.nn.sigmoid(gates[:, D:2 * D])
    gg = jnp.tanh(gates[:, 2 * D:3 * D])
    og = jax.nn.sigmoid(gates[:, 3 * D:4 * D])
    c_new = fg * c0 + ig * gg
    return og * jnp.tanh(c_new)


def _pre_body(feat_ref, degp_ref, Wg_ref, Wih_ref, Whh_ref, bih_ref, bhh_ref,
              h0_ref, c0_ref, ys_ref):
    W_new = _lstm_w_new(Wg_ref[...], Wih_ref[...], Whh_ref[...],
                        bih_ref[...], bhh_ref[...], h0_ref[...], c0_ref[...])
    deg = jnp.sum(degp_ref[...], axis=0) + 1.0
    dis = lax.rsqrt(deg)
    xw = jnp.dot(feat_ref[...], W_new, preferred_element_type=jnp.float32)
    y = xw * dis[:, None]
    ys_ref[0] = y[:, :HD2]
    ys_ref[1] = y[:, HD2:]


def _post_body(hp_ref, ys_ref, degp_ref, Wlin_ref, blin_ref, z_ref):
    deg = jnp.sum(degp_ref[...], axis=0) + 1.0
    dis = lax.rsqrt(deg)
    s = jnp.concatenate([hp_ref[0] + ys_ref[0], hp_ref[1] + ys_ref[1]],
                        axis=1)
    h = dis[:, None] * s
    z = jnp.maximum(h, 0.0)
    dn = (((1,), (1,)), ((), ()))
    z_ref[...] = (lax.dot_general(z, Wlin_ref[...], dn,
                                  preferred_element_type=jnp.float32)
                  + blin_ref[0, :])


_BLK = 1024
_GRID = NP // _BLK


def _full(shape):
    return pl.BlockSpec(shape, lambda j: tuple(0 for _ in shape))


def _pre_call(feat_pad, degp, W_gcn, W_ih, W_hh, b_ih2, b_hh2, h0, c0):
    return pl.pallas_call(
        _pre_body,
        grid=(_GRID,),
        in_specs=[
            pl.BlockSpec((_BLK, D), lambda j: (j, 0)),
            pl.BlockSpec((NW, _BLK), lambda j: (0, j)),
            _full((D, D)), _full((4 * D, D)), _full((4 * D, D)),
            _full((1, 4 * D)), _full((1, 4 * D)),
            _full((D, D)), _full((D, D)),
        ],
        out_specs=pl.BlockSpec((NC, _BLK, HD2), lambda j: (0, j, 0)),
        out_shape=jax.ShapeDtypeStruct((NC, NP, HD2), jnp.float32),
    )(feat_pad, degp, W_gcn, W_ih, W_hh, b_ih2, b_hh2, h0, c0)


def _post_call(hp, ys, degp, W_lin, b_lin2):
    return pl.pallas_call(
        _post_body,
        grid=(_GRID,),
        in_specs=[
            pl.BlockSpec((NC, _BLK, HD2), lambda j: (0, j, 0)),
            pl.BlockSpec((NC, _BLK, HD2), lambda j: (0, j, 0)),
            pl.BlockSpec((NW, _BLK), lambda j: (0, j)),
            _full((D, D)), _full((1, D)),
        ],
        out_specs=pl.BlockSpec((_BLK, D), lambda j: (j, 0)),
        out_shape=jax.ShapeDtypeStruct((NP, D), jnp.float32),
    )(hp, ys, degp, W_lin, b_lin2)


def kernel(edge_index, node_feat, W_gcn, W_ih, W_hh, b_ih, b_hh, h0, c0,
           W_lin, b_lin):
    row, col = edge_index[0], edge_index[1]
    pad = EP - E
    rowp = jnp.concatenate([row, jnp.zeros((pad,), jnp.int32)]).reshape(EP // B, B)
    colp = jnp.concatenate([col, jnp.full((pad,), N, jnp.int32)]).reshape(EP // B, B)
    feat_pad = jnp.pad(node_feat, ((0, NP - N), (0, 0)))
    b_ih2 = b_ih.reshape(1, 4 * D)
    b_hh2 = b_hh.reshape(1, 4 * D)
    b_lin2 = b_lin.reshape(1, D)

    degp = _deg_kernel(colp)
    ys = _pre_call(feat_pad, degp, W_gcn, W_ih, W_hh, b_ih2, b_hh2, h0, c0)
    hp = _msg_kernel(ys.reshape(NC * NP, HD2), rowp, colp)
    z = _post_call(hp, ys, degp, W_lin, b_lin2)
    return z[:N]


# speedup vs baseline: 1.2796x; 1.2796x over previous
import functools

import jax
import jax.numpy as jnp
from jax import lax
from jax.experimental import pallas as pl
from jax.experimental.pallas import tpu as pltpu
from jax.experimental.pallas import tpu_sc as plsc

N = 10000
E = 320000
D = 128

NC, NS, L = 2, 16, 16
NW = NC * NS
B = 128
CH = 80
EP = NW * CH * B
NP = 10240
RPT = NP // NS
HD2 = D // 2


_sc_mesh = plsc.VectorSubcoreMesh(core_axis_name="c", subcore_axis_name="s")
_sc_params = pltpu.CompilerParams(needs_layout_passes=False,
                                  use_tc_tiling_on_sc=False)


@functools.partial(
    pl.kernel,
    out_type=jax.ShapeDtypeStruct((NW, NP), jnp.float32),
    mesh=_sc_mesh,
    compiler_params=_sc_params,
    scratch_types=[
        pltpu.VMEM((CH, B), jnp.int32),
        pltpu.VMEM((NP,), jnp.float32),
    ],
)
def _deg_kernel(colp_hbm, degp_hbm, col_v, deg_v):
    cid = lax.axis_index("c")
    sid = lax.axis_index("s")
    wid = cid * NS + sid

    pltpu.sync_copy(colp_hbm.at[pl.ds(wid * CH, CH)], col_v)

    zeros16 = jnp.zeros((L,), jnp.float32)

    @pl.loop(0, NP // L)
    def _(i):
        deg_v[pl.ds(i * L, L)] = zeros16

    ones16 = jnp.ones((L,), jnp.float32)

    @pl.loop(0, CH)
    def _(i):
        for k in range(B // L):
            idx = col_v[i, pl.ds(k * L, L)]
            plsc.addupdate_scatter(deg_v, [idx], ones16)

    pltpu.sync_copy(deg_v, degp_hbm.at[wid])


CHT = EP // (NS * B)
CHP = 40
NB = 4


@functools.partial(
    pl.kernel,
    out_type=jax.ShapeDtypeStruct((NC, NP, HD2), jnp.float32),
    mesh=_sc_mesh,
    compiler_params=_sc_params,
    scratch_types=[
        pltpu.VMEM((CHP, B), jnp.int32),
        pltpu.VMEM((CHP, B), jnp.int32),
        [pltpu.VMEM((B, HD2), jnp.float32)] * NB,
        pltpu.VMEM((16, HD2), jnp.float32),
        pltpu.VMEM_SHARED((NP, HD2), jnp.float32),
        pltpu.VMEM_SHARED((NP, HD2), jnp.float32),
        [pltpu.SemaphoreType.DMA] * NB,
        [pltpu.SemaphoreType.DMA] * NB,
    ],
)
def _msg_kernel(ys_hbm, rowp_hbm, colp_hbm, hp_hbm,
                row_v, col_v, bufs, zbuf, y_s, acc_s, gsem, ssem):
    cid = lax.axis_index("c")
    sid = lax.axis_index("s")

    pltpu.sync_copy(ys_hbm.at[cid, pl.ds(sid * RPT, RPT)],
                    y_s.at[pl.ds(sid * RPT, RPT)])

    zeros16 = jnp.zeros((L,), jnp.float32)

    @pl.loop(0, 16)
    def _(i):
        for k in range(HD2 // L):
            zbuf[i, pl.ds(k * L, L)] = zeros16

    @pl.loop(0, RPT // 16)
    def _(j):
        pltpu.sync_copy(zbuf, acc_s.at[pl.ds(sid * RPT + j * 16, 16)])

    plsc.subcore_barrier()

    def one_pass(base):
        pltpu.sync_copy(rowp_hbm.at[pl.ds(base, CHP)], row_v)
        pltpu.sync_copy(colp_hbm.at[pl.ds(base, CHP)], col_v)
        for b in range(NB):
            pltpu.async_copy(y_s.at[row_v.at[b]], bufs[b], gsem[b])

        @pl.loop(0, CHP, step=NB)
        def _(j):
            for b in range(NB):
                pltpu.make_async_copy(y_s.at[row_v.at[j + b]],
                                      bufs[b], gsem[b]).wait()
                pltpu.async_copy(bufs[b], acc_s.at[col_v.at[j + b]],
                                 ssem[b], add=True)
            for b in range(NB):
                pltpu.make_async_copy(bufs[b], acc_s.at[col_v.at[j + b]],
                                      ssem[b]).wait()

                @pl.when(j + NB + b < CHP)
                def _():
                    pltpu.async_copy(y_s.at[row_v.at[j + NB + b]],
                                     bufs[b], gsem[b])

    for p in range(CHT // CHP):
        one_pass(sid * CHT + p * CHP)

    plsc.subcore_barrier()
    pltpu.sync_copy(acc_s.at[pl.ds(sid * RPT, RPT)],
                    hp_hbm.at[cid, pl.ds(sid * RPT, RPT)])


def _lstm_w_new(W_gcn, W_ih, W_hh, b_ih, b_hh, h0, c0):
    dn = (((1,), (1,)), ((), ()))
    gates = (lax.dot_general(W_gcn, W_ih, dn, preferred_element_type=jnp.float32)
             + lax.dot_general(h0, W_hh, dn, preferred_element_type=jnp.float32)
             + b_ih[0, :] + b_hh[0, :])
    ig = jax.nn.sigmoid(gates[:, 0:D])
    fg = jax.nn.sigmoid(gates[:, D:2 * D])
    gg = jnp.tanh(gates[:, 2 * D:3 * D])
    og = jax.nn.sigmoid(gates[:, 3 * D:4 * D])
    c_new = fg * c0 + ig * gg
    return og * jnp.tanh(c_new)


def _pre_body(feat_ref, degp_ref, Wg_ref, Wih_ref, Whh_ref, bih_ref, bhh_ref,
              h0_ref, c0_ref, ys_ref):
    W_new = _lstm_w_new(Wg_ref[...], Wih_ref[...], Whh_ref[...],
                        bih_ref[...], bhh_ref[...], h0_ref[...], c0_ref[...])
    deg = jnp.sum(degp_ref[...], axis=0) + 1.0
    dis = lax.rsqrt(deg)
    xw = jnp.dot(feat_ref[...], W_new, preferred_element_type=jnp.float32)
    y = xw * dis[:, None]
    ys_ref[0] = y[:, :HD2]
    ys_ref[1] = y[:, HD2:]


def _post_body(hp_ref, ys_ref, degp_ref, Wlin_ref, blin_ref, z_ref):
    deg = jnp.sum(degp_ref[...], axis=0) + 1.0
    dis = lax.rsqrt(deg)
    s = jnp.concatenate([hp_ref[0] + ys_ref[0], hp_ref[1] + ys_ref[1]],
                        axis=1)
    h = dis[:, None] * s
    z = jnp.maximum(h, 0.0)
    dn = (((1,), (1,)), ((), ()))
    z_ref[...] = (lax.dot_general(z, Wlin_ref[...], dn,
                                  preferred_element_type=jnp.float32)
                  + blin_ref[0, :])


_BLK = 1024
_GRID = NP // _BLK


def _full(shape):
    return pl.BlockSpec(shape, lambda j: tuple(0 for _ in shape))


def _pre_call(feat_pad, degp, W_gcn, W_ih, W_hh, b_ih2, b_hh2, h0, c0):
    return pl.pallas_call(
        _pre_body,
        grid=(_GRID,),
        in_specs=[
            pl.BlockSpec((_BLK, D), lambda j: (j, 0)),
            pl.BlockSpec((NW, _BLK), lambda j: (0, j)),
            _full((D, D)), _full((4 * D, D)), _full((4 * D, D)),
            _full((1, 4 * D)), _full((1, 4 * D)),
            _full((D, D)), _full((D, D)),
        ],
        out_specs=pl.BlockSpec((NC, _BLK, HD2), lambda j: (0, j, 0)),
        out_shape=jax.ShapeDtypeStruct((NC, NP, HD2), jnp.float32),
    )(feat_pad, degp, W_gcn, W_ih, W_hh, b_ih2, b_hh2, h0, c0)


def _post_call(hp, ys, degp, W_lin, b_lin2):
    return pl.pallas_call(
        _post_body,
        grid=(_GRID,),
        in_specs=[
            pl.BlockSpec((NC, _BLK, HD2), lambda j: (0, j, 0)),
            pl.BlockSpec((NC, _BLK, HD2), lambda j: (0, j, 0)),
            pl.BlockSpec((NW, _BLK), lambda j: (0, j)),
            _full((D, D)), _full((1, D)),
        ],
        out_specs=pl.BlockSpec((_BLK, D), lambda j: (j, 0)),
        out_shape=jax.ShapeDtypeStruct((NP, D), jnp.float32),
    )(hp, ys, degp, W_lin, b_lin2)


def kernel(edge_index, node_feat, W_gcn, W_ih, W_hh, b_ih, b_hh, h0, c0,
           W_lin, b_lin):
    row, col = edge_index[0], edge_index[1]
    pad = EP - E
    rowp = jnp.concatenate([row, jnp.zeros((pad,), jnp.int32)]).reshape(EP // B, B)
    colp = jnp.concatenate([col, jnp.full((pad,), N, jnp.int32)]).reshape(EP // B, B)
    feat_pad = jnp.pad(node_feat, ((0, NP - N), (0, 0)))
    b_ih2 = b_ih.reshape(1, 4 * D)
    b_hh2 = b_hh.reshape(1, 4 * D)
    b_lin2 = b_lin.reshape(1, D)

    degp = _deg_kernel(colp)
    ys = _pre_call(feat_pad, degp, W_gcn, W_ih, W_hh, b_ih2, b_hh2, h0, c0)
    hp = _msg_kernel(ys, rowp, colp)
    z = _post_call(hp, ys, degp, W_lin, b_lin2)
    return z[:N]
